# baseline (speedup 1.0000x reference)
import math
from functools import partial

import jax
import jax.numpy as jnp
from jax.experimental import pallas as pl
from jax.experimental.pallas import tpu as pltpu

_BN_EPS = 1e-5
_SLOPE = 0.1
_VMEM_LIMIT = 100 * 1024 * 1024


def _pick_tile(m):
    for t in (1024, 512, 256, 128, 64, 32, 16, 8):
        if m % t == 0:
            return t
    return m


def _fused_kernel(x_ref, w_ref, g_ref, b_ref, o_ref,
                  z_ref, sum_ref, sq_ref, scale_ref, shift_ref, *, tm, m):
    p = pl.program_id(0)
    i = pl.program_id(1)

    @pl.when(p == 0)
    def _compute():
        @pl.when(i == 0)
        def _init():
            sum_ref[...] = jnp.zeros_like(sum_ref)
            sq_ref[...] = jnp.zeros_like(sq_ref)

        xb = x_ref[...].astype(jnp.bfloat16)
        z = jnp.dot(xb, w_ref[...], preferred_element_type=jnp.float32)
        z_ref[pl.ds(i * tm, tm), :] = z.astype(jnp.bfloat16)
        sum_ref[...] += jnp.sum(z, axis=0, keepdims=True)
        sq_ref[...] += jnp.sum(z * z, axis=0, keepdims=True)

    @pl.when(p == 1)
    def _normalize():
        @pl.when(i == 0)
        def _fold_stats():
            inv_m = 1.0 / m
            mean = sum_ref[...] * inv_m
            var = jnp.maximum(sq_ref[...] * inv_m - mean * mean, 0.0)
            scale_ref[...] = g_ref[...] * jax.lax.rsqrt(var + _BN_EPS)
            shift_ref[...] = b_ref[...] - mean * scale_ref[...]

        zt = z_ref[pl.ds(i * tm, tm), :].astype(jnp.float32)
        y = zt * scale_ref[...] + shift_ref[...]
        o_ref[...] = jnp.where(y > 0, y, _SLOPE * y)


@jax.jit
def _run(x, weight, gamma, beta):
    B, N, in_dim = x.shape
    out_dim = weight.shape[0]
    M = B * N
    x2 = x.reshape(M, in_dim)
    wt = weight.T.astype(jnp.bfloat16)

    tm = _pick_tile(M)
    n_t = M // tm
    f32 = jnp.float32

    out = pl.pallas_call(
        partial(_fused_kernel, tm=tm, m=M),
        out_shape=jax.ShapeDtypeStruct((M, out_dim), x.dtype),
        grid=(2, n_t),
        in_specs=[pl.BlockSpec((tm, in_dim), lambda p, i: ((1 - p) * i, 0)),
                  pl.BlockSpec((in_dim, out_dim), lambda p, i: (0, 0)),
                  pl.BlockSpec((1, out_dim), lambda p, i: (0, 0)),
                  pl.BlockSpec((1, out_dim), lambda p, i: (0, 0))],
        out_specs=pl.BlockSpec((tm, out_dim), lambda p, i: (p * i, 0)),
        scratch_shapes=[pltpu.VMEM((M, out_dim), jnp.bfloat16),
                        pltpu.VMEM((1, out_dim), f32),
                        pltpu.VMEM((1, out_dim), f32),
                        pltpu.VMEM((1, out_dim), f32),
                        pltpu.VMEM((1, out_dim), f32)],
        compiler_params=pltpu.CompilerParams(
            dimension_semantics=("arbitrary", "arbitrary"),
            vmem_limit_bytes=_VMEM_LIMIT),
    )(x2, wt, gamma.reshape(1, out_dim).astype(f32),
      beta.reshape(1, out_dim).astype(f32))

    return out.reshape(B, N, out_dim)


def kernel(x, weight, bias, gamma, beta):
    del bias
    return _run(x, weight, gamma, beta)


# speedup vs baseline: 1.2661x; 1.2661x over previous
import math
from functools import partial

import jax
import jax.numpy as jnp
from jax.experimental import pallas as pl
from jax.experimental.pallas import tpu as pltpu

_BN_EPS = 1e-5
_SLOPE = 0.1
_VMEM_LIMIT = 100 * 1024 * 1024


def _pick_tile(m):
    for t in (2048, 1024, 512, 256, 128, 64, 32, 16, 8):
        if m % t == 0:
            return t
    return m


def _fused_kernel(x_ref, w_ref, g_ref, b_ref, o_ref,
                  z_ref, sum_ref, sq_ref, scale_ref, shift_ref,
                  *, tm0, tm1, n0, m):
    i = pl.program_id(0)

    @pl.when(i < n0)
    def _compute():
        @pl.when(i == 0)
        def _init():
            sum_ref[...] = jnp.zeros_like(sum_ref)
            sq_ref[...] = jnp.zeros_like(sq_ref)

        xb = x_ref[...].astype(jnp.bfloat16)
        z = jnp.dot(xb, w_ref[...], preferred_element_type=jnp.float32)
        z_ref[pl.ds(i * tm0, tm0), :] = z.astype(jnp.bfloat16)
        sum_ref[...] += jnp.sum(z, axis=0, keepdims=True)
        sq_ref[...] += jnp.sum(z * z, axis=0, keepdims=True)

    @pl.when(i >= n0)
    def _normalize():
        @pl.when(i == n0)
        def _fold_stats():
            inv_m = 1.0 / m
            mean = sum_ref[...] * inv_m
            var = jnp.maximum(sq_ref[...] * inv_m - mean * mean, 0.0)
            scale_ref[...] = g_ref[...] * jax.lax.rsqrt(var + _BN_EPS)
            shift_ref[...] = b_ref[...] - mean * scale_ref[...]

        j = i - n0
        zt = z_ref[pl.ds(j * tm1, tm1), :].astype(jnp.float32)
        y = zt * scale_ref[...] + shift_ref[...]
        o_ref[...] = jnp.where(y > 0, y, _SLOPE * y)


@jax.jit
def _run(x, weight, gamma, beta):
    B, N, in_dim = x.shape
    out_dim = weight.shape[0]
    M = B * N
    x2 = x.reshape(M, in_dim)
    wt = weight.T.astype(jnp.bfloat16)

    tm0 = _pick_tile(M)
    tm1 = 2 * tm0 if M % (2 * tm0) == 0 else tm0
    n0 = M // tm0
    n1 = M // tm1
    f32 = jnp.float32

    out = pl.pallas_call(
        partial(_fused_kernel, tm0=tm0, tm1=tm1, n0=n0, m=M),
        out_shape=jax.ShapeDtypeStruct((M, out_dim), x.dtype),
        grid=(n0 + n1,),
        in_specs=[pl.BlockSpec((tm0, in_dim),
                               lambda i: (jnp.minimum(i, n0 - 1), 0)),
                  pl.BlockSpec((in_dim, out_dim), lambda i: (0, 0)),
                  pl.BlockSpec((1, out_dim), lambda i: (0, 0)),
                  pl.BlockSpec((1, out_dim), lambda i: (0, 0))],
        out_specs=pl.BlockSpec((tm1, out_dim),
                               lambda i: (jnp.maximum(i - n0, 0), 0)),
        scratch_shapes=[pltpu.VMEM((M, out_dim), jnp.bfloat16),
                        pltpu.VMEM((1, out_dim), f32),
                        pltpu.VMEM((1, out_dim), f32),
                        pltpu.VMEM((1, out_dim), f32),
                        pltpu.VMEM((1, out_dim), f32)],
        compiler_params=pltpu.CompilerParams(
            dimension_semantics=("arbitrary",),
            vmem_limit_bytes=_VMEM_LIMIT),
    )(x2, wt, gamma.reshape(1, out_dim).astype(f32),
      beta.reshape(1, out_dim).astype(f32))

    return out.reshape(B, N, out_dim)


def kernel(x, weight, bias, gamma, beta):
    del bias
    return _run(x, weight, gamma, beta)


# speedup vs baseline: 1.3187x; 1.0416x over previous
import math
from functools import partial

import jax
import jax.numpy as jnp
from jax.experimental import pallas as pl
from jax.experimental.pallas import tpu as pltpu

_BN_EPS = 1e-5
_SLOPE = 0.1
_VMEM_LIMIT = 100 * 1024 * 1024


def _pick_tile(m):
    for t in (2048, 1024, 512, 256, 128, 64, 32, 16, 8):
        if m % t == 0:
            return t
    return m


def _fused_kernel(xa_ref, xb_ref, w_ref, g_ref, b_ref, o_ref,
                  z_ref, sum_ref, sq_ref, scale_ref, shift_ref,
                  *, tm0, tm1, n_pairs, n0, m):
    i = pl.program_id(0)

    @pl.when(i < n_pairs)
    def _compute():
        @pl.when(i == 0)
        def _init():
            sum_ref[...] = jnp.zeros_like(sum_ref)
            sq_ref[...] = jnp.zeros_like(sq_ref)

        w = w_ref[...]
        za = jnp.dot(xa_ref[...].astype(jnp.bfloat16), w,
                     preferred_element_type=jnp.float32)
        zb = jnp.dot(xb_ref[...].astype(jnp.bfloat16), w,
                     preferred_element_type=jnp.float32)
        z_ref[pl.ds((2 * i) * tm0, tm0), :] = za.astype(jnp.bfloat16)
        z_ref[pl.ds((2 * i + 1) * tm0, tm0), :] = zb.astype(jnp.bfloat16)
        sum_ref[...] += (jnp.sum(za, axis=0, keepdims=True) +
                         jnp.sum(zb, axis=0, keepdims=True))
        sq_ref[...] += (jnp.sum(za * za, axis=0, keepdims=True) +
                        jnp.sum(zb * zb, axis=0, keepdims=True))

    @pl.when(i >= n_pairs)
    def _normalize():
        @pl.when(i == n_pairs)
        def _fold_stats():
            inv_m = 1.0 / m
            mean = sum_ref[...] * inv_m
            var = jnp.maximum(sq_ref[...] * inv_m - mean * mean, 0.0)
            scale_ref[...] = g_ref[...] * jax.lax.rsqrt(var + _BN_EPS)
            shift_ref[...] = b_ref[...] - mean * scale_ref[...]

        j = i - n_pairs
        zt = z_ref[pl.ds(j * tm1, tm1), :].astype(jnp.float32)
        y = zt * scale_ref[...] + shift_ref[...]
        o_ref[...] = jnp.where(y > 0, y, _SLOPE * y)


@jax.jit
def _run(x, weight, gamma, beta):
    B, N, in_dim = x.shape
    out_dim = weight.shape[0]
    M = B * N
    x2 = x.reshape(M, in_dim)
    wt = weight.T.astype(jnp.bfloat16)

    tm0 = _pick_tile(M)
    if M % (2 * tm0) != 0:
        tm0 = tm0 // 2
    n_pairs = M // (2 * tm0)
    tm1 = tm0
    n1 = M // tm1
    f32 = jnp.float32

    def xa_map(i):
        return (2 * jnp.minimum(i, n_pairs - 1), 0)

    def xb_map(i):
        return (2 * jnp.minimum(i, n_pairs - 1) + 1, 0)

    out = pl.pallas_call(
        partial(_fused_kernel, tm0=tm0, tm1=tm1, n_pairs=n_pairs,
                n0=2 * n_pairs, m=M),
        out_shape=jax.ShapeDtypeStruct((M, out_dim), x.dtype),
        grid=(n_pairs + n1,),
        in_specs=[pl.BlockSpec((tm0, in_dim), xa_map),
                  pl.BlockSpec((tm0, in_dim), xb_map),
                  pl.BlockSpec((in_dim, out_dim), lambda i: (0, 0)),
                  pl.BlockSpec((1, out_dim), lambda i: (0, 0)),
                  pl.BlockSpec((1, out_dim), lambda i: (0, 0))],
        out_specs=pl.BlockSpec((tm1, out_dim),
                               lambda i: (jnp.maximum(i - n_pairs, 0), 0)),
        scratch_shapes=[pltpu.VMEM((M, out_dim), jnp.bfloat16),
                        pltpu.VMEM((1, out_dim), f32),
                        pltpu.VMEM((1, out_dim), f32),
                        pltpu.VMEM((1, out_dim), f32),
                        pltpu.VMEM((1, out_dim), f32)],
        compiler_params=pltpu.CompilerParams(
            dimension_semantics=("arbitrary",),
            vmem_limit_bytes=_VMEM_LIMIT),
    )(x2, x2, wt, gamma.reshape(1, out_dim).astype(f32),
      beta.reshape(1, out_dim).astype(f32))

    return out.reshape(B, N, out_dim)


def kernel(x, weight, bias, gamma, beta):
    del bias
    return _run(x, weight, gamma, beta)
